# Initial kernel scaffold; baseline (speedup 1.0000x reference)
import functools

import jax
import jax.numpy as jnp
from jax import lax
from jax.experimental import pallas as pl
from jax.experimental.pallas import tpu as pltpu
from jax.experimental.pallas import tpu_sc as plsc

NN = 10000
EE = 320000
NP = 10240
BLK = 1024
GRID = NP // BLK
EDG = EE + NN
NW = 32
CHUNK = 128
EPW = ((EDG + NW * CHUNK - 1) // (NW * CHUNK)) * CHUNK
NCH = EPW // CHUNK
EPAD = EPW * NW
ROWS_PT = NP // 16


def _tab1_body(x_ref, w1_ref, asm_ref, adm_ref, a1_ref, d1_ref, m1_ref):
    i = pl.program_id(0)
    xp = jnp.dot(x_ref[...], w1_ref[...], preferred_element_type=jnp.float32)
    asrc = jnp.dot(xp, asm_ref[...], preferred_element_type=jnp.float32)
    adst = jnp.dot(xp, adm_ref[...], preferred_element_type=jnp.float32)
    z8 = jnp.zeros((BLK, 8), jnp.float32)
    a1_ref[...] = jnp.concatenate([xp, asrc, z8], axis=1)
    d1_ref[...] = jnp.concatenate([adst, z8], axis=1)
    bs = jnp.concatenate([jnp.max(asrc, axis=0), jnp.zeros((8,), jnp.float32)])
    bd = jnp.concatenate([jnp.max(adst, axis=0), jnp.zeros((8,), jnp.float32)])
    blockm = jnp.concatenate(
        [bs[None, :], bd[None, :], jnp.zeros((6, 16), jnp.float32)], axis=0)

    @pl.when(i == 0)
    def _():
        m1_ref[...] = blockm

    @pl.when(i > 0)
    def _():
        m1_ref[...] = jnp.maximum(m1_ref[...], blockm)


_tab1 = pl.pallas_call(
    _tab1_body,
    grid=(GRID,),
    in_specs=[
        pl.BlockSpec((BLK, 128), lambda i: (i, 0)),
        pl.BlockSpec((128, 64), lambda i: (0, 0)),
        pl.BlockSpec((64, 8), lambda i: (0, 0)),
        pl.BlockSpec((64, 8), lambda i: (0, 0)),
    ],
    out_specs=[
        pl.BlockSpec((BLK, 80), lambda i: (i, 0)),
        pl.BlockSpec((BLK, 16), lambda i: (i, 0)),
        pl.BlockSpec((8, 16), lambda i: (0, 0)),
    ],
    out_shape=[
        jax.ShapeDtypeStruct((NP, 80), jnp.float32),
        jax.ShapeDtypeStruct((NP, 16), jnp.float32),
        jax.ShapeDtypeStruct((8, 16), jnp.float32),
    ],
)


def _edge1(src_hbm, dst_hbm, a1_hbm, d1_hbm, m1_hbm, z_hbm, out_hbm,
           acc_sh, idx_s, idx_d, arows, drows, msg, ms_v, md_v, sem):
    core = lax.axis_index("c")
    sub = lax.axis_index("s")
    wid = core * 16 + sub
    pltpu.sync_copy(z_hbm.at[pl.ds(sub * ROWS_PT, ROWS_PT)],
                    acc_sh.at[pl.ds(sub * ROWS_PT, ROWS_PT)])
    pltpu.sync_copy(m1_hbm.at[0], ms_v)
    pltpu.sync_copy(m1_hbm.at[1], md_v)
    plsc.subcore_barrier()
    m = ms_v[...] + md_v[...]
    shift = jnp.where(m >= 0.0, m, 0.2 * m)
    lane = lax.iota(jnp.int32, 16)
    headmask = lane < 8
    idxbase = (lane >= 8).astype(jnp.int32)
    base_w = wid * EPW

    def chunk_body(t, carry):
        base = base_w + t * CHUNK
        pltpu.sync_copy(src_hbm.at[pl.ds(base, CHUNK)], idx_s)
        pltpu.sync_copy(dst_hbm.at[pl.ds(base, CHUNK)], idx_d)
        pltpu.async_copy(a1_hbm.at[idx_s], arows, sem).wait()
        pltpu.async_copy(d1_hbm.at[idx_d], drows, sem).wait()

        def edge_body(i, c2):
            av = arows[i, pl.ds(64, 16)]
            dv = drows[i, pl.ds(0, 16)]
            e = av + dv
            e = jnp.where(e >= 0.0, e, 0.2 * e)
            w = jnp.exp(e - shift)
            w = jnp.where(headmask, w, 0.0)
            msg[i, pl.ds(64, 16)] = w
            for j in range(4):
                xv = arows[i, pl.ds(j * 16, 16)]
                wbc = jnp.take(w, idxbase + 2 * j, mode="promise_in_bounds")
                msg[i, pl.ds(j * 16, 16)] = xv * wbc
            return c2

        lax.fori_loop(0, CHUNK, edge_body, 0)
        pltpu.sync_copy(msg, acc_sh.at[idx_d], add=True)
        return carry

    lax.fori_loop(0, NCH, chunk_body, 0)
    plsc.subcore_barrier()
    pltpu.sync_copy(acc_sh.at[pl.ds(sub * ROWS_PT, ROWS_PT)],
                    out_hbm.at[core, pl.ds(sub * ROWS_PT, ROWS_PT)])


_mesh = plsc.VectorSubcoreMesh(core_axis_name="c", subcore_axis_name="s")

_edge1_call = functools.partial(
    pl.kernel,
    mesh=_mesh,
    out_type=jax.ShapeDtypeStruct((2, NP, 80), jnp.float32),
    scratch_types=[
        pltpu.VMEM_SHARED((NP, 80), jnp.float32),
        pltpu.VMEM((CHUNK,), jnp.int32),
        pltpu.VMEM((CHUNK,), jnp.int32),
        pltpu.VMEM((CHUNK, 80), jnp.float32),
        pltpu.VMEM((CHUNK, 16), jnp.float32),
        pltpu.VMEM((CHUNK, 80), jnp.float32),
        pltpu.VMEM((16,), jnp.float32),
        pltpu.VMEM((16,), jnp.float32),
        pltpu.SemaphoreType.DMA,
    ],
)(_edge1)


def _tab2_body(p0_ref, p1_ref, b1_ref, w2_ref, r_ref, as2_ref, ad2_ref,
               t2_ref, d2_ref, m2_ref):
    i = pl.program_id(0)
    p0 = p0_ref[...]
    p1 = p1_ref[...]
    num = p0[:, :64] + p1[:, :64]
    den = p0[:, 64:72] + p1[:, 64:72]
    denr = jnp.dot(den, r_ref[...], preferred_element_type=jnp.float32)
    h = num / (denr + 1e-16) + b1_ref[...]
    h = jnp.where(h > 0.0, h, jnp.expm1(h))
    h2 = jnp.dot(h, w2_ref[...], preferred_element_type=jnp.float32)
    s2 = jnp.dot(h2, as2_ref[...], preferred_element_type=jnp.float32)
    d2 = jnp.dot(h2, ad2_ref[...], preferred_element_type=jnp.float32)
    t2_ref[...] = jnp.concatenate(
        [h2, s2, jnp.zeros((BLK, 7), jnp.float32)], axis=1)
    d2_ref[...] = jnp.concatenate(
        [d2, jnp.zeros((BLK, 15), jnp.float32)], axis=1)
    blockm = jnp.concatenate(
        [jnp.full((1, 16), jnp.max(s2), jnp.float32),
         jnp.full((1, 16), jnp.max(d2), jnp.float32),
         jnp.zeros((6, 16), jnp.float32)], axis=0)

    @pl.when(i == 0)
    def _():
        m2_ref[...] = blockm

    @pl.when(i > 0)
    def _():
        m2_ref[...] = jnp.maximum(m2_ref[...], blockm)


_tab2 = pl.pallas_call(
    _tab2_body,
    grid=(GRID,),
    in_specs=[
        pl.BlockSpec((BLK, 80), lambda i: (i, 0)),
        pl.BlockSpec((BLK, 80), lambda i: (i, 0)),
        pl.BlockSpec((1, 64), lambda i: (0, 0)),
        pl.BlockSpec((64, 40), lambda i: (0, 0)),
        pl.BlockSpec((8, 64), lambda i: (0, 0)),
        pl.BlockSpec((40, 1), lambda i: (0, 0)),
        pl.BlockSpec((40, 1), lambda i: (0, 0)),
    ],
    out_specs=[
        pl.BlockSpec((BLK, 48), lambda i: (i, 0)),
        pl.BlockSpec((BLK, 16), lambda i: (i, 0)),
        pl.BlockSpec((8, 16), lambda i: (0, 0)),
    ],
    out_shape=[
        jax.ShapeDtypeStruct((NP, 48), jnp.float32),
        jax.ShapeDtypeStruct((NP, 16), jnp.float32),
        jax.ShapeDtypeStruct((8, 16), jnp.float32),
    ],
)


def _edge2(src_hbm, dst_hbm, t2_hbm, d2_hbm, m2_hbm, z_hbm, out_hbm,
           acc_sh, idx_s, idx_d, trows, drows, msg, ms_v, md_v, sem):
    core = lax.axis_index("c")
    sub = lax.axis_index("s")
    wid = core * 16 + sub
    pltpu.sync_copy(z_hbm.at[pl.ds(sub * ROWS_PT, ROWS_PT)],
                    acc_sh.at[pl.ds(sub * ROWS_PT, ROWS_PT)])
    pltpu.sync_copy(m2_hbm.at[0], ms_v)
    pltpu.sync_copy(m2_hbm.at[1], md_v)
    plsc.subcore_barrier()
    m = ms_v[...] + md_v[...]
    shift = jnp.where(m >= 0.0, m, 0.2 * m)
    lane = lax.iota(jnp.int32, 16)
    l8 = lane < 8
    eq8 = lane == 8
    idx8 = jnp.full((16,), 8, jnp.int32)
    idx0 = jnp.zeros((16,), jnp.int32)
    base_w = wid * EPW

    def chunk_body(t, carry):
        base = base_w + t * CHUNK
        pltpu.sync_copy(src_hbm.at[pl.ds(base, CHUNK)], idx_s)
        pltpu.sync_copy(dst_hbm.at[pl.ds(base, CHUNK)], idx_d)
        pltpu.async_copy(t2_hbm.at[idx_s], trows, sem).wait()
        pltpu.async_copy(d2_hbm.at[idx_d], drows, sem).wait()

        def edge_body(i, c2):
            tv2 = trows[i, pl.ds(32, 16)]
            dv = drows[i, pl.ds(0, 16)]
            asb = jnp.take(tv2, idx8, mode="promise_in_bounds")
            adb = jnp.take(dv, idx0, mode="promise_in_bounds")
            e = asb + adb
            e = jnp.where(e >= 0.0, e, 0.2 * e)
            wv = jnp.exp(e - shift)
            msg[i, pl.ds(0, 16)] = trows[i, pl.ds(0, 16)] * wv
            msg[i, pl.ds(16, 16)] = trows[i, pl.ds(16, 16)] * wv
            msg[i, pl.ds(32, 16)] = jnp.where(
                l8, tv2 * wv, jnp.where(eq8, wv, 0.0))
            return c2

        lax.fori_loop(0, CHUNK, edge_body, 0)
        pltpu.sync_copy(msg, acc_sh.at[idx_d], add=True)
        return carry

    lax.fori_loop(0, NCH, chunk_body, 0)
    plsc.subcore_barrier()
    pltpu.sync_copy(acc_sh.at[pl.ds(sub * ROWS_PT, ROWS_PT)],
                    out_hbm.at[core, pl.ds(sub * ROWS_PT, ROWS_PT)])


_edge2_call = functools.partial(
    pl.kernel,
    mesh=_mesh,
    out_type=jax.ShapeDtypeStruct((2, NP, 48), jnp.float32),
    scratch_types=[
        pltpu.VMEM_SHARED((NP, 48), jnp.float32),
        pltpu.VMEM((CHUNK,), jnp.int32),
        pltpu.VMEM((CHUNK,), jnp.int32),
        pltpu.VMEM((CHUNK, 48), jnp.float32),
        pltpu.VMEM((CHUNK, 16), jnp.float32),
        pltpu.VMEM((CHUNK, 48), jnp.float32),
        pltpu.VMEM((16,), jnp.float32),
        pltpu.VMEM((16,), jnp.float32),
        pltpu.SemaphoreType.DMA,
    ],
)(_edge2)


def _final_body(p0_ref, p1_ref, b2_ref, o_ref):
    p0 = p0_ref[...]
    p1 = p1_ref[...]
    num = p0[:, :40] + p1[:, :40]
    den = p0[:, 40:41] + p1[:, 40:41]
    o = num / (den + 1e-16) + b2_ref[...]
    z = o - jnp.max(o, axis=1, keepdims=True)
    o_ref[...] = z - jnp.log(jnp.sum(jnp.exp(z), axis=1, keepdims=True))


_final = pl.pallas_call(
    _final_body,
    grid=(GRID,),
    in_specs=[
        pl.BlockSpec((BLK, 48), lambda i: (i, 0)),
        pl.BlockSpec((BLK, 48), lambda i: (i, 0)),
        pl.BlockSpec((1, 40), lambda i: (0, 0)),
    ],
    out_specs=pl.BlockSpec((BLK, 40), lambda i: (i, 0)),
    out_shape=jax.ShapeDtypeStruct((NP, 40), jnp.float32),
)


def kernel(x, edge_index, W1, a_src1, a_dst1, b1, W2, a_src2, a_dst2, b2):
    f32 = jnp.float32
    x = x.astype(f32)
    ei = edge_index.astype(jnp.int32)
    ar = jnp.arange(NN, dtype=jnp.int32)
    npad = EPAD - EDG
    src = jnp.concatenate([ei[0], ar, jnp.zeros((npad,), jnp.int32)])
    dst = jnp.concatenate([ei[1], ar, jnp.full((npad,), NN, jnp.int32)])
    x_pad = jnp.zeros((NP, 128), f32).at[:NN].set(x)
    eye8 = jnp.eye(8, dtype=f32)
    asm = (eye8[:, None, :] * a_src1.astype(f32)[:, :, None]).reshape(64, 8)
    adm = (eye8[:, None, :] * a_dst1.astype(f32)[:, :, None]).reshape(64, 8)
    rmat = jnp.repeat(eye8, 8, axis=1)
    z1 = jnp.zeros((NP, 80), f32)
    z2 = jnp.zeros((NP, 48), f32)

    a1, d1, m1 = _tab1(x_pad, W1.astype(f32), asm, adm)
    part1 = _edge1_call(src, dst, a1, d1, m1, z1)
    t2, d2, m2 = _tab2(part1[0], part1[1], b1.astype(f32).reshape(1, 64),
                       W2.astype(f32), rmat,
                       a_src2.astype(f32).reshape(40, 1),
                       a_dst2.astype(f32).reshape(40, 1))
    part2 = _edge2_call(src, dst, t2, d2, m2, z2)
    out = _final(part2[0], part2[1], b2.astype(f32).reshape(1, 40))
    return out[:NN]


# speedup vs baseline: 38.9265x; 38.9265x over previous
import functools

import jax
import jax.numpy as jnp
from jax import lax
from jax.experimental import pallas as pl
from jax.experimental.pallas import tpu as pltpu
from jax.experimental.pallas import tpu_sc as plsc

NN = 10000
EE = 320000
NP = 10240
BLK = 1024
GRID = NP // BLK
EDG = EE + NN
NW = 32
CHUNK = 128
EPW = ((EDG + NW * CHUNK - 1) // (NW * CHUNK)) * CHUNK
NCH = EPW // CHUNK
EPAD = EPW * NW
ROWS_ACC = 10112
ROWS_PT = ROWS_ACC // 16

_GDN = lax.GatherDimensionNumbers(
    offset_dims=(), collapsed_slice_dims=(0,), start_index_map=(0,))


def _dyn_gather(v, idx):
    return lax.gather(v, idx[:, None], _GDN, slice_sizes=(1,),
                      mode=lax.GatherScatterMode.PROMISE_IN_BOUNDS)


def _tab1_body(x_ref, w1_ref, asm_ref, adm_ref, a1_ref, m1_ref):
    i = pl.program_id(0)
    xp = jnp.dot(x_ref[...], w1_ref[...], preferred_element_type=jnp.float32)
    asrc = jnp.dot(xp, asm_ref[...], preferred_element_type=jnp.float32)
    adst = jnp.dot(xp, adm_ref[...], preferred_element_type=jnp.float32)
    a1_ref[...] = jnp.concatenate(
        [xp, asrc, adst, jnp.zeros((BLK, 48), jnp.float32)], axis=1)
    bs = jnp.concatenate([jnp.max(asrc, axis=0), jnp.zeros((8,), jnp.float32)])
    bd = jnp.concatenate([jnp.max(adst, axis=0), jnp.zeros((8,), jnp.float32)])
    blockm = jnp.concatenate(
        [bs[None, :], bd[None, :], jnp.zeros((6, 16), jnp.float32)], axis=0)

    @pl.when(i == 0)
    def _():
        m1_ref[...] = blockm

    @pl.when(i > 0)
    def _():
        m1_ref[...] = jnp.maximum(m1_ref[...], blockm)


_tab1 = pl.pallas_call(
    _tab1_body,
    grid=(GRID,),
    in_specs=[
        pl.BlockSpec((BLK, 128), lambda i: (i, 0)),
        pl.BlockSpec((128, 64), lambda i: (0, 0)),
        pl.BlockSpec((64, 8), lambda i: (0, 0)),
        pl.BlockSpec((64, 8), lambda i: (0, 0)),
    ],
    out_specs=[
        pl.BlockSpec((BLK, 128), lambda i: (i, 0)),
        pl.BlockSpec((8, 16), lambda i: (0, 0)),
    ],
    out_shape=[
        jax.ShapeDtypeStruct((NP, 128), jnp.float32),
        jax.ShapeDtypeStruct((8, 16), jnp.float32),
    ],
)


def _edge1(src_hbm, dst_hbm, a1_hbm, m1_hbm, z_hbm, out_hbm,
           acc_sh, idx_s, idx_d, arows, drows, msg, ms_v, md_v,
           shift_v, sem):
    core = lax.axis_index("c")
    sub = lax.axis_index("s")
    wid = core * 16 + sub
    pltpu.sync_copy(z_hbm.at[pl.ds(sub * ROWS_PT, ROWS_PT)],
                    acc_sh.at[pl.ds(sub * ROWS_PT, ROWS_PT)])
    pltpu.sync_copy(z_hbm.at[pl.ds(0, CHUNK)], msg)
    pltpu.sync_copy(m1_hbm.at[0], ms_v)
    pltpu.sync_copy(m1_hbm.at[1], md_v)
    plsc.subcore_barrier()
    m = ms_v[...] + md_v[...]
    shift_v[...] = jnp.where(m >= 0.0, m, 0.2 * m)
    base_w = wid * EPW

    def chunk_body(t, carry):
        base = base_w + t * CHUNK
        pltpu.sync_copy(src_hbm.at[pl.ds(base, CHUNK)], idx_s)
        pltpu.sync_copy(dst_hbm.at[pl.ds(base, CHUNK)], idx_d)
        pltpu.async_copy(a1_hbm.at[idx_s], arows, sem).wait()
        pltpu.async_copy(a1_hbm.at[idx_d], drows, sem).wait()

        def edge_body(i, c2):
            lane = lax.iota(jnp.int32, 16)
            sv = arows[i, pl.ds(64, 16)]
            dv = drows[i, pl.ds(64, 16)]
            drot = _dyn_gather(dv, (lane + 8) & 15)
            e = sv + drot
            e = jnp.where(e >= 0.0, e, 0.2 * e)
            w = jnp.exp(e - shift_v[...])
            w = jnp.where(lane < 8, w, 0.0)
            msg[i, pl.ds(64, 16)] = w
            for j in range(4):
                xv = arows[i, pl.ds(j * 16, 16)]
                wbc = _dyn_gather(w, jnp.where(lane < 8, 2 * j, 2 * j + 1))
                msg[i, pl.ds(j * 16, 16)] = xv * wbc
            return c2

        lax.fori_loop(0, CHUNK, edge_body, 0)
        pltpu.sync_copy(msg, acc_sh.at[idx_d], add=True)
        return carry

    lax.fori_loop(0, NCH, chunk_body, 0)
    plsc.subcore_barrier()
    pltpu.sync_copy(acc_sh.at[pl.ds(sub * ROWS_PT, ROWS_PT)],
                    out_hbm.at[core, pl.ds(sub * ROWS_PT, ROWS_PT)])


_mesh = plsc.VectorSubcoreMesh(core_axis_name="c", subcore_axis_name="s")

_edge1_call = functools.partial(
    pl.kernel,
    mesh=_mesh,
    out_type=jax.ShapeDtypeStruct((2, NP, 128), jnp.float32),
    scratch_types=[
        pltpu.VMEM_SHARED((ROWS_ACC, 128), jnp.float32),
        pltpu.VMEM((CHUNK,), jnp.int32),
        pltpu.VMEM((CHUNK,), jnp.int32),
        pltpu.VMEM((CHUNK, 128), jnp.float32),
        pltpu.VMEM((CHUNK, 128), jnp.float32),
        pltpu.VMEM((CHUNK, 128), jnp.float32),
        pltpu.VMEM((16,), jnp.float32),
        pltpu.VMEM((16,), jnp.float32),
        pltpu.VMEM((16,), jnp.float32),
        pltpu.SemaphoreType.DMA,
    ],
)(_edge1)


def _tab2_body(p0_ref, p1_ref, b1_ref, w2_ref, r_ref, as2_ref, ad2_ref,
               t2_ref, m2_ref):
    i = pl.program_id(0)
    p0 = p0_ref[...]
    p1 = p1_ref[...]
    num = p0[:, :64] + p1[:, :64]
    den = p0[:, 64:72] + p1[:, 64:72]
    denr = jnp.dot(den, r_ref[...], preferred_element_type=jnp.float32)
    h = num / (denr + 1e-16) + b1_ref[...]
    h = jnp.where(h > 0.0, h, jnp.exp(jnp.minimum(h, 0.0)) - 1.0)
    h2 = jnp.dot(h, w2_ref[...], preferred_element_type=jnp.float32)
    s2 = jnp.dot(h2, as2_ref[...], preferred_element_type=jnp.float32)
    d2 = jnp.dot(h2, ad2_ref[...], preferred_element_type=jnp.float32)
    t2_ref[...] = jnp.concatenate(
        [h2, s2, jnp.zeros((BLK, 7), jnp.float32),
         d2, jnp.zeros((BLK, 79), jnp.float32)], axis=1)
    rowid = lax.broadcasted_iota(jnp.int32, (BLK, 1), 0) + i * BLK
    valid = rowid < NN
    s2m = jnp.where(valid, s2, -jnp.inf)
    d2m = jnp.where(valid, d2, -jnp.inf)
    blockm = jnp.concatenate(
        [jnp.full((1, 16), jnp.max(s2m), jnp.float32),
         jnp.full((1, 16), jnp.max(d2m), jnp.float32),
         jnp.zeros((6, 16), jnp.float32)], axis=0)

    @pl.when(i == 0)
    def _():
        m2_ref[...] = blockm

    @pl.when(i > 0)
    def _():
        m2_ref[...] = jnp.maximum(m2_ref[...], blockm)


_tab2 = pl.pallas_call(
    _tab2_body,
    grid=(GRID,),
    in_specs=[
        pl.BlockSpec((BLK, 128), lambda i: (i, 0)),
        pl.BlockSpec((BLK, 128), lambda i: (i, 0)),
        pl.BlockSpec((1, 64), lambda i: (0, 0)),
        pl.BlockSpec((64, 40), lambda i: (0, 0)),
        pl.BlockSpec((8, 64), lambda i: (0, 0)),
        pl.BlockSpec((40, 1), lambda i: (0, 0)),
        pl.BlockSpec((40, 1), lambda i: (0, 0)),
    ],
    out_specs=[
        pl.BlockSpec((BLK, 128), lambda i: (i, 0)),
        pl.BlockSpec((8, 16), lambda i: (0, 0)),
    ],
    out_shape=[
        jax.ShapeDtypeStruct((NP, 128), jnp.float32),
        jax.ShapeDtypeStruct((8, 16), jnp.float32),
    ],
)


def _edge2(src_hbm, dst_hbm, t2_hbm, m2_hbm, z_hbm, out_hbm,
           acc_sh, idx_s, idx_d, trows, drows, msg, ms_v, md_v, shift_v, sem):
    core = lax.axis_index("c")
    sub = lax.axis_index("s")
    wid = core * 16 + sub
    pltpu.sync_copy(z_hbm.at[pl.ds(sub * ROWS_PT, ROWS_PT)],
                    acc_sh.at[pl.ds(sub * ROWS_PT, ROWS_PT)])
    pltpu.sync_copy(z_hbm.at[pl.ds(0, CHUNK)], msg)
    pltpu.sync_copy(m2_hbm.at[0], ms_v)
    pltpu.sync_copy(m2_hbm.at[1], md_v)
    plsc.subcore_barrier()
    m = ms_v[...] + md_v[...]
    shift_v[...] = jnp.where(m >= 0.0, m, 0.2 * m)
    base_w = wid * EPW

    def chunk_body(t, carry):
        base = base_w + t * CHUNK
        pltpu.sync_copy(src_hbm.at[pl.ds(base, CHUNK)], idx_s)
        pltpu.sync_copy(dst_hbm.at[pl.ds(base, CHUNK)], idx_d)
        pltpu.async_copy(t2_hbm.at[idx_s], trows, sem).wait()
        pltpu.async_copy(t2_hbm.at[idx_d], drows, sem).wait()

        def edge_body(i, c2):
            lane = lax.iota(jnp.int32, 16)
            tv2 = trows[i, pl.ds(32, 16)]
            dv = drows[i, pl.ds(48, 16)]
            asb = _dyn_gather(tv2, jnp.full((16,), 8, jnp.int32))
            adb = _dyn_gather(dv, jnp.zeros((16,), jnp.int32))
            e = asb + adb
            e = jnp.where(e >= 0.0, e, 0.2 * e)
            wv = jnp.exp(e - shift_v[...])
            msg[i, pl.ds(0, 16)] = trows[i, pl.ds(0, 16)] * wv
            msg[i, pl.ds(16, 16)] = trows[i, pl.ds(16, 16)] * wv
            msg[i, pl.ds(32, 16)] = jnp.where(
                lane < 8, tv2 * wv, jnp.where(lane == 8, wv, 0.0))
            return c2

        lax.fori_loop(0, CHUNK, edge_body, 0)
        pltpu.sync_copy(msg, acc_sh.at[idx_d], add=True)
        return carry

    lax.fori_loop(0, NCH, chunk_body, 0)
    plsc.subcore_barrier()
    pltpu.sync_copy(acc_sh.at[pl.ds(sub * ROWS_PT, ROWS_PT)],
                    out_hbm.at[core, pl.ds(sub * ROWS_PT, ROWS_PT)])


_edge2_call = functools.partial(
    pl.kernel,
    mesh=_mesh,
    out_type=jax.ShapeDtypeStruct((2, NP, 128), jnp.float32),
    scratch_types=[
        pltpu.VMEM_SHARED((ROWS_ACC, 128), jnp.float32),
        pltpu.VMEM((CHUNK,), jnp.int32),
        pltpu.VMEM((CHUNK,), jnp.int32),
        pltpu.VMEM((CHUNK, 128), jnp.float32),
        pltpu.VMEM((CHUNK, 128), jnp.float32),
        pltpu.VMEM((CHUNK, 128), jnp.float32),
        pltpu.VMEM((16,), jnp.float32),
        pltpu.VMEM((16,), jnp.float32),
        pltpu.VMEM((16,), jnp.float32),
        pltpu.SemaphoreType.DMA,
    ],
)(_edge2)


def _final_body(p0_ref, p1_ref, b2_ref, o_ref):
    p0 = p0_ref[...]
    p1 = p1_ref[...]
    num = p0[:, :40] + p1[:, :40]
    den = p0[:, 40:41] + p1[:, 40:41]
    o = num / (den + 1e-16) + b2_ref[...]
    z = o - jnp.max(o, axis=1, keepdims=True)
    o_ref[...] = z - jnp.log(jnp.sum(jnp.exp(z), axis=1, keepdims=True))


_final = pl.pallas_call(
    _final_body,
    grid=(GRID,),
    in_specs=[
        pl.BlockSpec((BLK, 128), lambda i: (i, 0)),
        pl.BlockSpec((BLK, 128), lambda i: (i, 0)),
        pl.BlockSpec((1, 40), lambda i: (0, 0)),
    ],
    out_specs=pl.BlockSpec((BLK, 40), lambda i: (i, 0)),
    out_shape=jax.ShapeDtypeStruct((NP, 40), jnp.float32),
)


def kernel(x, edge_index, W1, a_src1, a_dst1, b1, W2, a_src2, a_dst2, b2):
    f32 = jnp.float32
    x = x.astype(f32)
    ei = edge_index.astype(jnp.int32)
    ar = jnp.arange(NN, dtype=jnp.int32)
    npad = EPAD - EDG
    src = jnp.concatenate([ei[0], ar, jnp.zeros((npad,), jnp.int32)])
    dst = jnp.concatenate([ei[1], ar, jnp.full((npad,), NN, jnp.int32)])
    x_pad = jnp.zeros((NP, 128), f32).at[:NN].set(x)
    eye8 = jnp.eye(8, dtype=f32)
    asm = (eye8[:, None, :] * a_src1.astype(f32)[:, :, None]).reshape(64, 8)
    adm = (eye8[:, None, :] * a_dst1.astype(f32)[:, :, None]).reshape(64, 8)
    rmat = jnp.repeat(eye8, 8, axis=1)
    zz = jnp.zeros((NP, 128), f32)

    a1, m1 = _tab1(x_pad, W1.astype(f32), asm, adm)
    part1 = _edge1_call(src, dst, a1, m1, zz)
    t2, m2 = _tab2(part1[0], part1[1], b1.astype(f32).reshape(1, 64),
                   W2.astype(f32), rmat,
                   a_src2.astype(f32).reshape(40, 1),
                   a_dst2.astype(f32).reshape(40, 1))
    part2 = _edge2_call(src, dst, t2, m2, zz)
    out = _final(part2[0], part2[1], b2.astype(f32).reshape(1, 40))
    return out[:NN]
